# Initial kernel scaffold; baseline (speedup 1.0000x reference)
import jax
import jax.numpy as jnp
from jax.experimental import pallas as pl


def kernel(bits, comp_table, flip_table):
    raise NotImplementedError("write your pallas kernel here")


# speedup vs baseline: 287.3249x; 287.3249x over previous
import jax
import jax.numpy as jnp
import numpy as np
from jax import lax
from jax.experimental import pallas as pl
from jax.experimental.pallas import tpu as pltpu
from jax.experimental.pallas import tpu_sc as plsc

INPUT_BITS = 64
N_GROUPS = 4
GROUP_BITS = 16
TABLE = 1 << GROUP_BITS

NC, NS = 2, 16
NW = NC * NS
TILES_PER_GROUP = NW // N_GROUPS
CH = 8192

_PACK_BLK = 4096
_BLK = 2048


def _pack_kernel(ct_ref, sw_ref, o_ref):
    ct = ct_ref[...].astype(jnp.bfloat16)
    sw = sw_ref[...]
    o_ref[...] = lax.dot_general(
        ct, sw, (((1,), (0,)), ((), ())),
        preferred_element_type=jnp.float32).astype(jnp.int32)


def _addr_kernel(b_ref, wt_ref, o_ref):
    bits = b_ref[...].astype(jnp.bfloat16)
    wt = wt_ref[...]
    a = lax.dot_general(wt, bits, (((1,), (1,)), ((), ())),
                        preferred_element_type=jnp.float32)
    o_ref[...] = a.astype(jnp.int32)


def _final_kernel(a_ref, g_ref, ft_ref, eye_ref, o_ref):
    blk = a_ref.shape[1]
    a_t = a_ref[...]
    g_t = g_ref[...]

    def rep16(x):
        return jnp.concatenate(
            [jnp.broadcast_to(x[g:g + 1, :], (GROUP_BITS, blk))
             for g in range(N_GROUPS)], axis=0)

    k = lax.broadcasted_iota(jnp.int32, (INPUT_BITS, blk), 0) & 15
    u = (rep16(a_t) >> k) & 1
    comp = (rep16(g_t) >> k) & 1

    ft = ft_ref[...]
    fpack = jnp.zeros((INPUT_BITS, 1), jnp.int32)
    for aa in range(8):
        fpack = fpack | (ft[:, aa:aa + 1] << aa)

    z = jnp.zeros((1, blk), jnp.int32)
    prev = jnp.concatenate([z, u[:INPUT_BITS - 1, :]], axis=0)
    nxt = jnp.concatenate([u[1:, :], z], axis=0)
    ctx = prev * 4 + u * 2 + nxt
    flip = (jnp.broadcast_to(fpack, (INPUT_BITS, blk)) >> ctx) & 1

    o_t = (comp ^ u ^ flip).astype(jnp.bfloat16)
    o = lax.dot_general(o_t, eye_ref[...], (((0,), (0,)), ((), ())),
                        preferred_element_type=jnp.float32)
    o_ref[...] = o.astype(jnp.int32)


def _make_sc_gather(batch):
    rows_per_tile = batch // TILES_PER_GROUP

    def body(ptab_hbm, addr_hbm, out_hbm, tab_v, idx_v, res_v):
        cid = lax.axis_index("c")
        sid = lax.axis_index("s")
        wid = sid * NC + cid
        g = wid % N_GROUPS
        t = wid // N_GROUPS
        base = t * rows_per_tile
        pltpu.sync_copy(ptab_hbm.at[g], tab_v)

        for c in range(rows_per_tile // CH):
            off = base + c * CH
            pltpu.sync_copy(addr_hbm.at[g, pl.ds(off, CH)], idx_v)

            def gather_step(i, carry):
                ix = idx_v[pl.ds(i * 16, 16)]
                res_v[pl.ds(i * 16, 16)] = plsc.load_gather(tab_v, [ix])
                return carry

            lax.fori_loop(0, CH // 16, gather_step, 0)
            pltpu.sync_copy(res_v, out_hbm.at[g, pl.ds(off, CH)])

    mesh = plsc.VectorSubcoreMesh(core_axis_name="c", subcore_axis_name="s",
                                  num_cores=NC, num_subcores=NS)
    return pl.kernel(
        body,
        out_type=jax.ShapeDtypeStruct((N_GROUPS, batch), jnp.int32),
        mesh=mesh,
        compiler_params=pltpu.CompilerParams(needs_layout_passes=False),
        scratch_types=[
            pltpu.VMEM((TABLE,), jnp.int32),
            pltpu.VMEM((CH,), jnp.int32),
            pltpu.VMEM((CH,), jnp.int32),
        ],
    )


def kernel(bits, comp_table, flip_table):
    batch = bits.shape[0]

    n_rows = (N_GROUPS * TABLE * GROUP_BITS) // 128
    ct2d = comp_table.reshape(n_rows, 128)
    sw = np.zeros((128, 8), np.float32)
    for l in range(128):
        sw[l, l // GROUP_BITS] = float(1 << (l % GROUP_BITS))
    sw = jnp.asarray(sw, jnp.bfloat16)
    packed = pl.pallas_call(
        _pack_kernel,
        grid=(n_rows // _PACK_BLK,),
        in_specs=[
            pl.BlockSpec((_PACK_BLK, 128), lambda i: (i, 0)),
            pl.BlockSpec((128, 8), lambda i: (0, 0)),
        ],
        out_specs=pl.BlockSpec((_PACK_BLK, 8), lambda i: (i, 0)),
        out_shape=jax.ShapeDtypeStruct((n_rows, 8), jnp.int32),
    )(ct2d, sw)
    ptab = packed.reshape(N_GROUPS, TABLE)

    wt = np.zeros((N_GROUPS, INPUT_BITS), np.float32)
    for j in range(INPUT_BITS):
        wt[j // GROUP_BITS, j] = float(1 << (j % GROUP_BITS))
    wt = jnp.asarray(wt, jnp.bfloat16)
    n_blk = batch // _BLK
    addr_t = pl.pallas_call(
        _addr_kernel,
        grid=(n_blk,),
        in_specs=[
            pl.BlockSpec((_BLK, INPUT_BITS), lambda i: (i, 0)),
            pl.BlockSpec((N_GROUPS, INPUT_BITS), lambda i: (0, 0)),
        ],
        out_specs=pl.BlockSpec((N_GROUPS, _BLK), lambda i: (0, i)),
        out_shape=jax.ShapeDtypeStruct((N_GROUPS, batch), jnp.int32),
    )(bits, wt)

    gath_t = _make_sc_gather(batch)(ptab, addr_t)

    eye = jnp.asarray(np.eye(INPUT_BITS, dtype=np.float32), jnp.bfloat16)
    out = pl.pallas_call(
        _final_kernel,
        grid=(n_blk,),
        in_specs=[
            pl.BlockSpec((N_GROUPS, _BLK), lambda i: (0, i)),
            pl.BlockSpec((N_GROUPS, _BLK), lambda i: (0, i)),
            pl.BlockSpec((INPUT_BITS, 8), lambda i: (0, 0)),
            pl.BlockSpec((INPUT_BITS, INPUT_BITS), lambda i: (0, 0)),
        ],
        out_specs=pl.BlockSpec((_BLK, INPUT_BITS), lambda i: (i, 0)),
        out_shape=jax.ShapeDtypeStruct((batch, INPUT_BITS), jnp.int32),
    )(addr_t, gath_t, flip_table, eye)
    return out
